# baseline (speedup 1.0000x reference)
import numpy as np

import jax
import jax.numpy as jnp
from jax import lax
from jax.experimental import pallas as pl
from jax.experimental.pallas import tpu as pltpu
from jax.experimental.pallas import tpu_sc as plsc

B = 16384
F = 6
H = 16
L = 16
NS = 16
ROWS_PER_W = B // NS
CHUNKS = ROWS_PER_W // L
UNROLL = 8
MED_RANK = B // 2
WPACK = F * H + H + H + L

LN2 = np.float32(0.6931471805599453)

_MESH = plsc.VectorSubcoreMesh(
    core_axis_name="c", subcore_axis_name="s", num_cores=1)
_PARAMS = pltpu.CompilerParams(needs_layout_passes=False)


def _body(stats, wb, out, st_v, wb_v, row_v, cnt_v, shared, sem):
    sid = lax.axis_index("s")
    base = sid * ROWS_PER_W
    copies = [
        pltpu.async_copy(stats.at[pl.ds(j * B + base, ROWS_PER_W)],
                         st_v.at[pl.ds(j * ROWS_PER_W, ROWS_PER_W)], sem)
        for j in range(F)
    ]
    copies.append(pltpu.async_copy(wb, wb_v, sem))
    for c in copies:
        c.wait()

    w1rows = [wb_v[pl.ds(j * H, H)] for j in range(F)]
    b1vec = wb_v[pl.ds(F * H, H)]
    w2vec = wb_v[pl.ds(F * H + H, H)]
    w1s = [[w1rows[j][i] for i in range(H)] for j in range(F)]
    b1s = [b1vec[i] for i in range(H)]
    w2s = [w2vec[i] for i in range(H)]
    b2s = wb_v[pl.ds(F * H + 2 * H, L)][0]

    def one_vreg(row0):
        f = [st_v[pl.ds(j * ROWS_PER_W + row0, L)] for j in range(F)]
        logit = jnp.full((L,), b2s, dtype=jnp.float32)
        for i in range(H):
            h = b1s[i] + w1s[0][i] * f[0]
            for j in range(1, F):
                h = h + w1s[j][i] * f[j]
            h = jnp.maximum(h, 0.0)
            logit = logit + w2s[i] * h
        return logit

    def chunk(c, carry):
        acc2, acc3 = carry
        logits = [one_vreg(c * (UNROLL * L) + u * L) for u in range(UNROLL)]
        for lg in logits:
            acc2 = acc2 + jnp.where(lg < -LN2, 1.0, 0.0)
            acc3 = acc3 + jnp.where(lg < LN2, 1.0, 0.0)
        return acc2, acc3

    zero = jnp.zeros((L,), jnp.float32)
    acc2, acc3 = lax.fori_loop(0, CHUNKS // UNROLL, chunk, (zero, zero))
    c2 = jnp.sum(acc2)
    c3 = jnp.sum(acc3)
    lane = jnp.arange(L, dtype=jnp.int32)
    row_v[...] = jnp.where(lane == 0, c2, jnp.where(lane == 1, c3, 0.0))
    pltpu.sync_copy(row_v, shared.at[pl.ds(sid * L, L)])
    plsc.subcore_barrier()

    @pl.when(sid == 0)
    def _():
        pltpu.sync_copy(shared, cnt_v)
        acc = cnt_v[pl.ds(0, L)]
        for w in range(1, NS):
            acc = acc + cnt_v[pl.ds(w * L, L)]
        tot2 = jnp.sum(jnp.where(lane == 0, acc, 0.0))
        tot3 = jnp.sum(jnp.where(lane == 1, acc, 0.0))
        k = (1.0 + jnp.where(tot2 < float(MED_RANK), 1.0, 0.0)
                 + jnp.where(tot3 < float(MED_RANK), 1.0, 0.0))
        row_v[...] = jnp.full((L,), k, dtype=jnp.float32)
        pltpu.sync_copy(row_v, out)


_selector = pl.kernel(
    _body,
    out_type=jax.ShapeDtypeStruct((L,), jnp.float32),
    mesh=_MESH,
    scratch_types=[
        pltpu.VMEM((F * ROWS_PER_W,), jnp.float32),
        pltpu.VMEM((WPACK,), jnp.float32),
        pltpu.VMEM((L,), jnp.float32),
        pltpu.VMEM((NS * L,), jnp.float32),
        pltpu.VMEM_SHARED((NS * L,), jnp.float32),
        pltpu.SemaphoreType.DMA,
    ],
    compiler_params=_PARAMS,
)


def kernel(x, sparsity, variance, magnitude, norm, skewness, concentration,
           W1, b1, W2, b2):
    del x
    stats = jnp.concatenate([sparsity, variance, magnitude, norm,
                             skewness, concentration])
    wb = jnp.concatenate([W1.reshape(F * H), b1, W2.reshape(H),
                          jnp.broadcast_to(b2, (L,))])
    out16 = _selector(stats, wb)
    return out16[0]


# speedup vs baseline: 1.0197x; 1.0197x over previous
import numpy as np

import jax
import jax.numpy as jnp
from jax import lax
from jax.experimental import pallas as pl
from jax.experimental.pallas import tpu as pltpu
from jax.experimental.pallas import tpu_sc as plsc

B = 16384
F = 6
H = 16
L = 16
NS = 16
ROWS_PER_W = B // NS
CHUNKS = ROWS_PER_W // L
UNROLL = 8
MED_RANK = B // 2
WPACK = F * H + H + H + L

LN2 = np.float32(0.6931471805599453)

_MESH = plsc.VectorSubcoreMesh(
    core_axis_name="c", subcore_axis_name="s", num_cores=1)
_PARAMS = pltpu.CompilerParams(needs_layout_passes=False)


def _body(stats, wb, out, st_v, wb_v, row_v, cnt_v, shared, shared_w, sem):
    sid = lax.axis_index("s")
    base = sid * ROWS_PER_W
    copies = [
        pltpu.async_copy(stats.at[pl.ds(j * B + base, ROWS_PER_W)],
                         st_v.at[pl.ds(j * ROWS_PER_W, ROWS_PER_W)], sem)
        for j in range(F)
    ]
    @pl.when(sid == 0)
    def _():
        pltpu.sync_copy(wb, shared_w)
    plsc.subcore_barrier()
    pltpu.sync_copy(shared_w, wb_v)

    w1rows = [wb_v[pl.ds(j * H, H)] for j in range(F)]
    b1vec = wb_v[pl.ds(F * H, H)]
    w2vec = wb_v[pl.ds(F * H + H, H)]
    w1s = [[w1rows[j][i] for i in range(H)] for j in range(F)]
    b1s = [b1vec[i] for i in range(H)]
    w2s = [w2vec[i] for i in range(H)]
    b2s = wb_v[pl.ds(F * H + 2 * H, L)][0]

    for c in copies:
        c.wait()

    def one_vreg(row0):
        f = [st_v[pl.ds(j * ROWS_PER_W + row0, L)] for j in range(F)]
        logit = jnp.full((L,), b2s, dtype=jnp.float32)
        for i in range(H):
            h = b1s[i] + w1s[0][i] * f[0]
            for j in range(1, F):
                h = h + w1s[j][i] * f[j]
            h = jnp.maximum(h, 0.0)
            logit = logit + w2s[i] * h
        return logit

    def chunk(c, carry):
        acc2, acc3 = carry
        logits = [one_vreg(c * (UNROLL * L) + u * L) for u in range(UNROLL)]
        for lg in logits:
            acc2 = acc2 + jnp.where(lg < -LN2, 1.0, 0.0)
            acc3 = acc3 + jnp.where(lg < LN2, 1.0, 0.0)
        return acc2, acc3

    zero = jnp.zeros((L,), jnp.float32)
    acc2, acc3 = lax.fori_loop(0, CHUNKS // UNROLL, chunk, (zero, zero))
    c2 = jnp.sum(acc2)
    c3 = jnp.sum(acc3)
    lane = jnp.arange(L, dtype=jnp.int32)
    row_v[...] = jnp.where(lane == 0, c2, jnp.where(lane == 1, c3, 0.0))
    pltpu.sync_copy(row_v, shared.at[pl.ds(sid * L, L)])
    plsc.subcore_barrier()

    @pl.when(sid == 0)
    def _():
        pltpu.sync_copy(shared, cnt_v)
        acc = cnt_v[pl.ds(0, L)]
        for w in range(1, NS):
            acc = acc + cnt_v[pl.ds(w * L, L)]
        tot2 = jnp.sum(jnp.where(lane == 0, acc, 0.0))
        tot3 = jnp.sum(jnp.where(lane == 1, acc, 0.0))
        k = (1.0 + jnp.where(tot2 < float(MED_RANK), 1.0, 0.0)
                 + jnp.where(tot3 < float(MED_RANK), 1.0, 0.0))
        row_v[...] = jnp.full((L,), k, dtype=jnp.float32)
        pltpu.sync_copy(row_v, out)


_selector = pl.kernel(
    _body,
    out_type=jax.ShapeDtypeStruct((L,), jnp.float32),
    mesh=_MESH,
    scratch_types=[
        pltpu.VMEM((F * ROWS_PER_W,), jnp.float32),
        pltpu.VMEM((WPACK,), jnp.float32),
        pltpu.VMEM((L,), jnp.float32),
        pltpu.VMEM((NS * L,), jnp.float32),
        pltpu.VMEM_SHARED((NS * L,), jnp.float32),
        pltpu.VMEM_SHARED((WPACK,), jnp.float32),
        pltpu.SemaphoreType.DMA,
    ],
    compiler_params=_PARAMS,
)


def kernel(x, sparsity, variance, magnitude, norm, skewness, concentration,
           W1, b1, W2, b2):
    del x
    stats = jnp.concatenate([sparsity, variance, magnitude, norm,
                             skewness, concentration])
    wb = jnp.concatenate([W1.reshape(F * H), b1, W2.reshape(H),
                          jnp.broadcast_to(b2, (L,))])
    out16 = _selector(stats, wb)
    return out16[0]
